# Initial kernel scaffold; baseline (speedup 1.0000x reference)
import functools

import jax
import jax.numpy as jnp
from jax import lax
from jax.experimental import pallas as pl
from jax.experimental.pallas import tpu as pltpu
from jax.experimental.pallas import tpu_sc as plsc

D = 128
H = 8
C = 16
PAY_W = D + C
NC = 2
NS = 16
NW = NC * NS
EDGE_BLK = 80
ROW_BLK = 1000


def _tc_prep(x, w_mat, a_src_m, a_dst_m):
  n = x.shape[0]

  def body(x_ref, w_ref, ms_ref, md_ref, pay_ref, adst_ref):
    xw = jnp.dot(x_ref[...], w_ref[...], preferred_element_type=jnp.float32)
    asrc = jnp.dot(xw, ms_ref[...], preferred_element_type=jnp.float32)
    adstv = jnp.dot(xw, md_ref[...], preferred_element_type=jnp.float32)
    pay_ref[...] = jnp.concatenate([xw, asrc], axis=1)
    adst_ref[...] = adstv

  return pl.pallas_call(
      body,
      grid=(n // ROW_BLK,),
      in_specs=[
          pl.BlockSpec((ROW_BLK, D), lambda i: (i, 0)),
          pl.BlockSpec((D, D), lambda i: (0, 0)),
          pl.BlockSpec((D, C), lambda i: (0, 0)),
          pl.BlockSpec((D, C), lambda i: (0, 0)),
      ],
      out_specs=[
          pl.BlockSpec((ROW_BLK, PAY_W), lambda i: (i, 0)),
          pl.BlockSpec((ROW_BLK, C), lambda i: (i, 0)),
      ],
      out_shape=[
          jax.ShapeDtypeStruct((n, PAY_W), jnp.float32),
          jax.ShapeDtypeStruct((n, C), jnp.float32),
      ],
  )(x, w_mat, a_src_m, a_dst_m)


def _sc_edge(src, dst, pay, adstt, zeros_tbl):
  e = src.shape[0]
  epw = e // NW
  nblk = epw // EDGE_BLK
  n = pay.shape[0]
  rpt = n // NS

  mesh = plsc.VectorSubcoreMesh(
      core_axis_name="c", subcore_axis_name="s",
      num_cores=NC, num_subcores=NS)

  @functools.partial(
      pl.kernel,
      out_type=jax.ShapeDtypeStruct((NC, n, PAY_W), jnp.float32),
      mesh=mesh,
      scratch_types=[
          pltpu.VMEM((EDGE_BLK,), jnp.int32),
          pltpu.VMEM((EDGE_BLK,), jnp.int32),
          pltpu.VMEM((EDGE_BLK, PAY_W), jnp.float32),
          pltpu.VMEM((EDGE_BLK, C), jnp.float32),
          pltpu.VMEM((EDGE_BLK, PAY_W), jnp.float32),
          pltpu.VMEM_SHARED((n, PAY_W), jnp.float32),
          pltpu.SemaphoreType.DMA,
          pltpu.SemaphoreType.DMA,
      ],
  )
  def k(src_hbm, dst_hbm, pay_hbm, adst_hbm, zero_hbm, out_hbm,
        sidx_v, didx_v, pay_v, adst_v, comb_v, acc, sem1, sem2):
    cid = lax.axis_index("c")
    sid = lax.axis_index("s")
    wid = sid * NC + cid

    r0 = sid * rpt
    pltpu.sync_copy(zero_hbm.at[pl.ds(r0, rpt)], acc.at[pl.ds(r0, rpt)])
    plsc.subcore_barrier()

    def block(b, carry):
      base = wid * epw + b * EDGE_BLK
      pltpu.sync_copy(src_hbm.at[pl.ds(base, EDGE_BLK)], sidx_v)
      pltpu.sync_copy(dst_hbm.at[pl.ds(base, EDGE_BLK)], didx_v)
      cp1 = pltpu.async_copy(pay_hbm.at[sidx_v], pay_v, sem1)
      cp2 = pltpu.async_copy(adst_hbm.at[didx_v], adst_v, sem2)
      cp1.wait()
      cp2.wait()

      def edge(i, carry2):
        a = pay_v[i, pl.ds(D, C)] + adst_v[i, :]
        w = jnp.exp(jnp.maximum(a, 0.2 * a))
        comb_v[i, pl.ds(D, C)] = w
        for h in range(H):
          wb = lax.broadcast(comb_v[i, D + h], (C,))
          comb_v[i, pl.ds(h * C, C)] = pay_v[i, pl.ds(h * C, C)] * wb
        return carry2

      lax.fori_loop(0, EDGE_BLK, edge, 0)
      pltpu.sync_copy(comb_v, acc.at[didx_v], add=True)
      return carry

    lax.fori_loop(0, nblk, block, 0)

    plsc.subcore_barrier()
    pltpu.sync_copy(acc.at[pl.ds(r0, rpt)],
                    out_hbm.at[cid, pl.ds(r0, rpt)])

  return k(src, dst, pay, adstt, zeros_tbl)


def _tc_final(acc0, acc1, pay, adstt, bias_row, sel):
  n = pay.shape[0]

  def body(a0, a1, payr, adr, br, selr, out_ref):
    num = a0[:, :D] + a1[:, :D]
    den8 = a0[:, D:D + H] + a1[:, D:D + H]
    aa = payr[:, D:D + H] + adr[:, :H]
    ws = jnp.exp(jnp.maximum(aa, 0.2 * aa))
    xw = payr[:, :D]
    ws128 = jnp.dot(ws, selr[...], preferred_element_type=jnp.float32)
    den128 = jnp.dot(den8 + ws, selr[...],
                     preferred_element_type=jnp.float32)
    out_ref[...] = (num + ws128 * xw) / (den128 + 1e-16) + br[...]

  return pl.pallas_call(
      body,
      grid=(n // ROW_BLK,),
      in_specs=[
          pl.BlockSpec((ROW_BLK, PAY_W), lambda i: (i, 0)),
          pl.BlockSpec((ROW_BLK, PAY_W), lambda i: (i, 0)),
          pl.BlockSpec((ROW_BLK, PAY_W), lambda i: (i, 0)),
          pl.BlockSpec((ROW_BLK, C), lambda i: (i, 0)),
          pl.BlockSpec((1, D), lambda i: (0, 0)),
          pl.BlockSpec((H, D), lambda i: (0, 0)),
      ],
      out_specs=pl.BlockSpec((ROW_BLK, D), lambda i: (i, 0)),
      out_shape=jax.ShapeDtypeStruct((n, D), jnp.float32),
  )(acc0, acc1, pay, adstt, bias_row, sel)


def kernel(x, edge_index, W, att_src, att_dst, bias):
  x = x.astype(jnp.float32)
  src = edge_index[0].astype(jnp.int32)
  dst = edge_index[1].astype(jnp.int32)

  eye = jnp.eye(H, dtype=jnp.float32)
  a_src_m = jnp.pad(
      (att_src.astype(jnp.float32)[:, :, None] * eye[:, None, :])
      .reshape(H * C, H), ((0, 0), (0, C - H)))
  a_dst_m = jnp.pad(
      (att_dst.astype(jnp.float32)[:, :, None] * eye[:, None, :])
      .reshape(H * C, H), ((0, 0), (0, C - H)))

  pay, adstt = _tc_prep(x, W.astype(jnp.float32), a_src_m, a_dst_m)
  zeros_tbl = jnp.zeros((x.shape[0], PAY_W), jnp.float32)
  acc = _sc_edge(src, dst, pay, adstt, zeros_tbl)

  sel = jnp.repeat(eye, C, axis=1)
  return _tc_final(acc[0], acc[1], pay, adstt,
                   bias.astype(jnp.float32).reshape(1, D), sel)


# speedup vs baseline: 68.4353x; 68.4353x over previous
import functools

import jax
import jax.numpy as jnp
from jax import lax
from jax.experimental import pallas as pl
from jax.experimental.pallas import tpu as pltpu
from jax.experimental.pallas import tpu_sc as plsc

D = 128
H = 8
C = 16
PAY_W = D + C
NC = 2
NS = 16
NW = NC * NS
EDGE_BLK = 80
ROW_BLK = 1000
ACC_PAD = NS * 8


def _tc_prep(x, w_mat, a_src_m, a_dst_m):
  n = x.shape[0]

  def body(x_ref, w_ref, ms_ref, md_ref, pay_ref, adst_ref):
    xw = jnp.dot(x_ref[...], w_ref[...], preferred_element_type=jnp.float32)
    asrc = jnp.dot(xw, ms_ref[...], preferred_element_type=jnp.float32)
    adstv = jnp.dot(xw, md_ref[...], preferred_element_type=jnp.float32)
    pay_ref[...] = jnp.concatenate([xw, asrc], axis=1)
    adst_ref[...] = adstv

  return pl.pallas_call(
      body,
      grid=(n // ROW_BLK,),
      in_specs=[
          pl.BlockSpec((ROW_BLK, D), lambda i: (i, 0)),
          pl.BlockSpec((D, D), lambda i: (0, 0)),
          pl.BlockSpec((D, C), lambda i: (0, 0)),
          pl.BlockSpec((D, C), lambda i: (0, 0)),
      ],
      out_specs=[
          pl.BlockSpec((ROW_BLK, PAY_W), lambda i: (i, 0)),
          pl.BlockSpec((ROW_BLK, C), lambda i: (i, 0)),
      ],
      out_shape=[
          jax.ShapeDtypeStruct((n, PAY_W), jnp.float32),
          jax.ShapeDtypeStruct((n, C), jnp.float32),
      ],
  )(x, w_mat, a_src_m, a_dst_m)


def _sc_edge(src, dst, pay, adstt, zeros_tbl):
  e = src.shape[0]
  epw = e // NW
  nblk = epw // EDGE_BLK
  npad = zeros_tbl.shape[0]
  rpt = npad // NS

  mesh = plsc.VectorSubcoreMesh(
      core_axis_name="c", subcore_axis_name="s",
      num_cores=NC, num_subcores=NS)

  @functools.partial(
      pl.kernel,
      out_type=jax.ShapeDtypeStruct((NC, npad, PAY_W), jnp.float32),
      mesh=mesh,
      compiler_params=pltpu.CompilerParams(use_tc_tiling_on_sc=False),
      scratch_types=[
          pltpu.VMEM((EDGE_BLK,), jnp.int32),
          pltpu.VMEM((EDGE_BLK,), jnp.int32),
          pltpu.VMEM((EDGE_BLK, PAY_W), jnp.float32),
          pltpu.VMEM((EDGE_BLK, C), jnp.float32),
          pltpu.VMEM((EDGE_BLK, PAY_W), jnp.float32),
          pltpu.VMEM_SHARED((npad, PAY_W), jnp.float32),
          pltpu.SemaphoreType.DMA,
          pltpu.SemaphoreType.DMA,
      ],
  )
  def k(src_hbm, dst_hbm, pay_hbm, adst_hbm, zero_hbm, out_hbm,
        sidx_v, didx_v, pay_v, adst_v, comb_v, acc, sem1, sem2):
    cid = lax.axis_index("c")
    sid = lax.axis_index("s")
    wid = sid * NC + cid

    r0 = sid * rpt
    pltpu.sync_copy(zero_hbm.at[pl.ds(r0, rpt)], acc.at[pl.ds(r0, rpt)])
    plsc.subcore_barrier()

    def block(b, carry):
      base = wid * epw + b * EDGE_BLK
      pltpu.sync_copy(src_hbm.at[pl.ds(base, EDGE_BLK)], sidx_v)
      pltpu.sync_copy(dst_hbm.at[pl.ds(base, EDGE_BLK)], didx_v)
      cp1 = pltpu.async_copy(pay_hbm.at[sidx_v], pay_v, sem1)
      cp2 = pltpu.async_copy(adst_hbm.at[didx_v], adst_v, sem2)
      cp1.wait()
      cp2.wait()

      def edge(i, carry2):
        a = pay_v[i, pl.ds(D, C)] + adst_v[i, :]
        w = jnp.exp(jnp.maximum(a, 0.2 * a))
        comb_v[i, pl.ds(D, C)] = w
        for h in range(H):
          wb = lax.broadcast(w[h], (C,))
          comb_v[i, pl.ds(h * C, C)] = pay_v[i, pl.ds(h * C, C)] * wb
        return carry2

      lax.fori_loop(0, EDGE_BLK, edge, 0)
      pltpu.sync_copy(comb_v, acc.at[didx_v], add=True)
      return carry

    lax.fori_loop(0, nblk, block, 0)

    plsc.subcore_barrier()
    pltpu.sync_copy(acc.at[pl.ds(r0, rpt)],
                    out_hbm.at[cid, pl.ds(r0, rpt)])

  return k(src, dst, pay, adstt, zeros_tbl)


def _tc_final(acc0, acc1, pay, adstt, bias_row, sel):
  n = pay.shape[0]

  def body(a0, a1, payr, adr, br, selr, out_ref):
    num = a0[:, :D] + a1[:, :D]
    den8 = a0[:, D:D + H] + a1[:, D:D + H]
    aa = payr[:, D:D + H] + adr[:, :H]
    ws = jnp.exp(jnp.maximum(aa, 0.2 * aa))
    xw = payr[:, :D]
    ws128 = jnp.dot(ws, selr[...], preferred_element_type=jnp.float32)
    den128 = jnp.dot(den8 + ws, selr[...],
                     preferred_element_type=jnp.float32)
    out_ref[...] = (num + ws128 * xw) / (den128 + 1e-16) + br[...]

  return pl.pallas_call(
      body,
      grid=(n // ROW_BLK,),
      in_specs=[
          pl.BlockSpec((ROW_BLK, PAY_W), lambda i: (i, 0)),
          pl.BlockSpec((ROW_BLK, PAY_W), lambda i: (i, 0)),
          pl.BlockSpec((ROW_BLK, PAY_W), lambda i: (i, 0)),
          pl.BlockSpec((ROW_BLK, C), lambda i: (i, 0)),
          pl.BlockSpec((1, D), lambda i: (0, 0)),
          pl.BlockSpec((H, D), lambda i: (0, 0)),
      ],
      out_specs=pl.BlockSpec((ROW_BLK, D), lambda i: (i, 0)),
      out_shape=jax.ShapeDtypeStruct((n, D), jnp.float32),
  )(acc0, acc1, pay, adstt, bias_row, sel)


def kernel(x, edge_index, W, att_src, att_dst, bias):
  x = x.astype(jnp.float32)
  src = edge_index[0].astype(jnp.int32)
  dst = edge_index[1].astype(jnp.int32)

  eye = jnp.eye(H, dtype=jnp.float32)
  a_src_m = jnp.pad(
      (att_src.astype(jnp.float32)[:, :, None] * eye[:, None, :])
      .reshape(H * C, H), ((0, 0), (0, C - H)))
  a_dst_m = jnp.pad(
      (att_dst.astype(jnp.float32)[:, :, None] * eye[:, None, :])
      .reshape(H * C, H), ((0, 0), (0, C - H)))

  pay, adstt = _tc_prep(x, W.astype(jnp.float32), a_src_m, a_dst_m)
  n = x.shape[0]
  npad = -(-n // (NS * 8)) * (NS * 8)
  zeros_tbl = jnp.zeros((npad, PAY_W), jnp.float32)
  acc = _sc_edge(src, dst, pay, adstt, zeros_tbl)

  sel = jnp.repeat(eye, C, axis=1)
  return _tc_final(acc[0, :n], acc[1, :n], pay, adstt,
                   bias.astype(jnp.float32).reshape(1, D), sel)


# speedup vs baseline: 110.5886x; 1.6160x over previous
import functools

import jax
import jax.numpy as jnp
from jax import lax
from jax.experimental import pallas as pl
from jax.experimental.pallas import tpu as pltpu
from jax.experimental.pallas import tpu_sc as plsc

D = 128
H = 8
C = 16
PAY_W = D + C
NC = 2
NS = 16
NW = NC * NS
EDGE_BLK = 80
ROW_BLK = 1000
ACC_PAD = NS * 8


def _tc_prep(x, w_mat, a_src_m, a_dst_m):
  n = x.shape[0]

  def body(x_ref, w_ref, ms_ref, md_ref, pay_ref, adst_ref):
    xw = jnp.dot(x_ref[...], w_ref[...], preferred_element_type=jnp.float32)
    asrc = jnp.dot(xw, ms_ref[...], preferred_element_type=jnp.float32)
    adstv = jnp.dot(xw, md_ref[...], preferred_element_type=jnp.float32)
    pay_ref[...] = jnp.concatenate([xw, asrc], axis=1)
    adst_ref[...] = adstv

  return pl.pallas_call(
      body,
      grid=(n // ROW_BLK,),
      in_specs=[
          pl.BlockSpec((ROW_BLK, D), lambda i: (i, 0)),
          pl.BlockSpec((D, D), lambda i: (0, 0)),
          pl.BlockSpec((D, C), lambda i: (0, 0)),
          pl.BlockSpec((D, C), lambda i: (0, 0)),
      ],
      out_specs=[
          pl.BlockSpec((ROW_BLK, PAY_W), lambda i: (i, 0)),
          pl.BlockSpec((ROW_BLK, C), lambda i: (i, 0)),
      ],
      out_shape=[
          jax.ShapeDtypeStruct((n, PAY_W), jnp.float32),
          jax.ShapeDtypeStruct((n, C), jnp.float32),
      ],
  )(x, w_mat, a_src_m, a_dst_m)


def _sc_edge(src, dst, pay, adstt, zeros_tbl):
  e = src.shape[0]
  epw = e // NW
  nblk = epw // EDGE_BLK
  npad = zeros_tbl.shape[0]
  rpt = npad // NS

  packed3 = (src | (dst << 16)).reshape(NW, nblk, EDGE_BLK)

  mesh = plsc.VectorSubcoreMesh(
      core_axis_name="c", subcore_axis_name="s",
      num_cores=NC, num_subcores=NS)

  @functools.partial(
      pl.kernel,
      out_type=jax.ShapeDtypeStruct((NC, npad, PAY_W), jnp.float32),
      mesh=mesh,
      compiler_params=pltpu.CompilerParams(use_tc_tiling_on_sc=False),
      scratch_types=[
          pltpu.VMEM((nblk, EDGE_BLK), jnp.int32),
          pltpu.VMEM((EDGE_BLK,), jnp.int32),
          pltpu.VMEM((EDGE_BLK,), jnp.int32),
          pltpu.VMEM((EDGE_BLK,), jnp.int32),
          pltpu.VMEM((EDGE_BLK,), jnp.int32),
          pltpu.VMEM((EDGE_BLK, PAY_W), jnp.float32),
          pltpu.VMEM((EDGE_BLK, PAY_W), jnp.float32),
          pltpu.VMEM((EDGE_BLK, C), jnp.float32),
          pltpu.VMEM((EDGE_BLK, C), jnp.float32),
          pltpu.VMEM_SHARED((npad, PAY_W), jnp.float32),
          pltpu.SemaphoreType.DMA,
          pltpu.SemaphoreType.DMA,
      ],
  )
  def k(packed_hbm, pay_hbm, adst_hbm, zero_hbm, out_hbm,
        packed_v, sidx0, didx0, sidx1, didx1, pay_v0, pay_v1,
        adst_v0, adst_v1, acc, semg0, semg1):
    cid = lax.axis_index("c")
    sid = lax.axis_index("s")
    wid = sid * NC + cid

    pltpu.sync_copy(packed_hbm.at[wid], packed_v)
    r0 = sid * rpt
    pltpu.sync_copy(zero_hbm.at[pl.ds(r0, rpt)], acc.at[pl.ds(r0, rpt)])
    plsc.subcore_barrier()

    def unpack(b, sidx_v, didx_v):
      for j in range(EDGE_BLK // C):
        v = packed_v[b, pl.ds(j * C, C)]
        sidx_v[pl.ds(j * C, C)] = v & 0xFFFF
        didx_v[pl.ds(j * C, C)] = v >> 16

    def fire(b, sidx_v, didx_v, pay_v, adst_v, sem):
      unpack(b, sidx_v, didx_v)
      pltpu.async_copy(pay_hbm.at[sidx_v], pay_v, sem)
      pltpu.async_copy(adst_hbm.at[didx_v], adst_v, sem)

    def drain(pay_v, adst_v, sem):
      pltpu.make_async_copy(pay_hbm.at[sidx0], pay_v, sem).wait()
      pltpu.make_async_copy(adst_hbm.at[didx0], adst_v, sem).wait()

    def compute(didx_v, pay_v, adst_v):
      def edge(i, carry2):
        a = pay_v[i, pl.ds(D, C)] + adst_v[i, :]
        w = jnp.exp(jnp.maximum(a, 0.2 * a))
        pay_v[i, pl.ds(D, C)] = w
        for h in range(H):
          wb = lax.broadcast(w[h], (C,))
          pay_v[i, pl.ds(h * C, C)] = pay_v[i, pl.ds(h * C, C)] * wb
        return carry2

      lax.fori_loop(0, EDGE_BLK, edge, 0)
      pltpu.sync_copy(pay_v, acc.at[didx_v], add=True)

    pairs = (nblk - 1) // 2
    fire(0, sidx0, didx0, pay_v0, adst_v0, semg0)

    def pair(t, carry):
      b0 = 2 * t
      fire(b0 + 1, sidx1, didx1, pay_v1, adst_v1, semg1)
      drain(pay_v0, adst_v0, semg0)
      compute(didx0, pay_v0, adst_v0)
      fire(b0 + 2, sidx0, didx0, pay_v0, adst_v0, semg0)
      drain(pay_v1, adst_v1, semg1)
      compute(didx1, pay_v1, adst_v1)
      return carry

    lax.fori_loop(0, pairs, pair, 0)

    if nblk - 2 * pairs == 2:
      fire(2 * pairs + 1, sidx1, didx1, pay_v1, adst_v1, semg1)
      drain(pay_v0, adst_v0, semg0)
      compute(didx0, pay_v0, adst_v0)
      drain(pay_v1, adst_v1, semg1)
      compute(didx1, pay_v1, adst_v1)
    else:
      drain(pay_v0, adst_v0, semg0)
      compute(didx0, pay_v0, adst_v0)

    plsc.subcore_barrier()
    pltpu.sync_copy(acc.at[pl.ds(r0, rpt)],
                    out_hbm.at[cid, pl.ds(r0, rpt)])

  return k(packed3, pay, adstt, zeros_tbl)


def _tc_final(acc0, acc1, pay, adstt, bias_row, sel):
  n = pay.shape[0]

  def body(a0, a1, payr, adr, br, selr, out_ref):
    num = a0[:, :D] + a1[:, :D]
    den8 = a0[:, D:D + H] + a1[:, D:D + H]
    aa = payr[:, D:D + H] + adr[:, :H]
    ws = jnp.exp(jnp.maximum(aa, 0.2 * aa))
    xw = payr[:, :D]
    ws128 = jnp.dot(ws, selr[...], preferred_element_type=jnp.float32)
    den128 = jnp.dot(den8 + ws, selr[...],
                     preferred_element_type=jnp.float32)
    out_ref[...] = (num + ws128 * xw) / (den128 + 1e-16) + br[...]

  return pl.pallas_call(
      body,
      grid=(n // ROW_BLK,),
      in_specs=[
          pl.BlockSpec((ROW_BLK, PAY_W), lambda i: (i, 0)),
          pl.BlockSpec((ROW_BLK, PAY_W), lambda i: (i, 0)),
          pl.BlockSpec((ROW_BLK, PAY_W), lambda i: (i, 0)),
          pl.BlockSpec((ROW_BLK, C), lambda i: (i, 0)),
          pl.BlockSpec((1, D), lambda i: (0, 0)),
          pl.BlockSpec((H, D), lambda i: (0, 0)),
      ],
      out_specs=pl.BlockSpec((ROW_BLK, D), lambda i: (i, 0)),
      out_shape=jax.ShapeDtypeStruct((n, D), jnp.float32),
  )(acc0, acc1, pay, adstt, bias_row, sel)


def kernel(x, edge_index, W, att_src, att_dst, bias):
  x = x.astype(jnp.float32)
  src = edge_index[0].astype(jnp.int32)
  dst = edge_index[1].astype(jnp.int32)

  eye = jnp.eye(H, dtype=jnp.float32)
  a_src_m = jnp.pad(
      (att_src.astype(jnp.float32)[:, :, None] * eye[:, None, :])
      .reshape(H * C, H), ((0, 0), (0, C - H)))
  a_dst_m = jnp.pad(
      (att_dst.astype(jnp.float32)[:, :, None] * eye[:, None, :])
      .reshape(H * C, H), ((0, 0), (0, C - H)))

  pay, adstt = _tc_prep(x, W.astype(jnp.float32), a_src_m, a_dst_m)
  n = x.shape[0]
  npad = -(-n // (NS * 8)) * (NS * 8)
  zeros_tbl = jnp.zeros((npad, PAY_W), jnp.float32)
  acc = _sc_edge(src, dst, pay, adstt, zeros_tbl)

  sel = jnp.repeat(eye, C, axis=1)
  return _tc_final(acc[0, :n], acc[1, :n], pay, adstt,
                   bias.astype(jnp.float32).reshape(1, D), sel)


# speedup vs baseline: 124.8195x; 1.1287x over previous
import functools

import jax
import jax.numpy as jnp
from jax import lax
from jax.experimental import pallas as pl
from jax.experimental.pallas import tpu as pltpu
from jax.experimental.pallas import tpu_sc as plsc

D = 128
H = 8
C = 16
PAY_W = D + C
NC = 2
NS = 16
NW = NC * NS
EDGE_BLK = 80
IDX_CHUNK = 5
ROW_BLK = 1000


def _tc_prep(x, w_mat, a_src_m, a_dst_m):
  n = x.shape[0]

  def body(x_ref, w_ref, ms_ref, md_ref, pay_ref, adst_ref):
    xw = jnp.dot(x_ref[...], w_ref[...], preferred_element_type=jnp.float32)
    asrc = jnp.dot(xw, ms_ref[...], preferred_element_type=jnp.float32)
    adstv = jnp.dot(xw, md_ref[...], preferred_element_type=jnp.float32)
    pay_ref[...] = jnp.concatenate([xw, asrc], axis=1)
    adst_ref[...] = adstv

  return pl.pallas_call(
      body,
      grid=(n // ROW_BLK,),
      in_specs=[
          pl.BlockSpec((ROW_BLK, D), lambda i: (i, 0)),
          pl.BlockSpec((D, D), lambda i: (0, 0)),
          pl.BlockSpec((D, C), lambda i: (0, 0)),
          pl.BlockSpec((D, C), lambda i: (0, 0)),
      ],
      out_specs=[
          pl.BlockSpec((ROW_BLK, PAY_W), lambda i: (i, 0)),
          pl.BlockSpec((ROW_BLK, C), lambda i: (i, 0)),
      ],
      out_shape=[
          jax.ShapeDtypeStruct((n, PAY_W), jnp.float32),
          jax.ShapeDtypeStruct((n, C), jnp.float32),
      ],
  )(x, w_mat, a_src_m, a_dst_m)


def _sc_edge(src, dst, pay, adstt, zeros_tbl):
  e = src.shape[0]
  epw = e // NW
  nblk = epw // EDGE_BLK
  npad = zeros_tbl.shape[0]
  rpt = npad // NS
  assert nblk % IDX_CHUNK == 0 and nblk >= 2

  packed3 = (src | (dst << 16)).reshape(NW, nblk, EDGE_BLK)

  mesh = plsc.VectorSubcoreMesh(
      core_axis_name="c", subcore_axis_name="s",
      num_cores=NC, num_subcores=NS)

  @functools.partial(
      pl.kernel,
      out_type=jax.ShapeDtypeStruct((NC, npad, PAY_W), jnp.float32),
      mesh=mesh,
      compiler_params=pltpu.CompilerParams(use_tc_tiling_on_sc=False),
      scratch_types=[
          pltpu.VMEM((IDX_CHUNK, EDGE_BLK), jnp.int32),
          [pltpu.VMEM((EDGE_BLK,), jnp.int32) for _ in range(3)],
          [pltpu.VMEM((EDGE_BLK,), jnp.int32) for _ in range(3)],
          [pltpu.VMEM((EDGE_BLK, PAY_W), jnp.float32) for _ in range(3)],
          [pltpu.VMEM((EDGE_BLK, C), jnp.float32) for _ in range(3)],
          pltpu.VMEM_SHARED((npad, PAY_W), jnp.float32),
          [pltpu.SemaphoreType.DMA for _ in range(3)],
          [pltpu.SemaphoreType.DMA for _ in range(3)],
      ],
  )
  def k(packed_hbm, pay_hbm, adst_hbm, zero_hbm, out_hbm,
        pk, sidx, didx, payb, adstb, acc, semg, sems):
    cid = lax.axis_index("c")
    sid = lax.axis_index("s")
    wid = sid * NC + cid

    r0 = sid * rpt
    pltpu.sync_copy(zero_hbm.at[pl.ds(r0, rpt)], acc.at[pl.ds(r0, rpt)])
    plsc.subcore_barrier()

    def refill(b):
      pltpu.sync_copy(packed_hbm.at[wid, pl.ds(b, IDX_CHUNK)], pk)

    def fire(b, s):
      bm = b % IDX_CHUNK
      for j in range(EDGE_BLK // C):
        v = pk[bm, pl.ds(j * C, C)]
        sidx[s][pl.ds(j * C, C)] = v & 0xFFFF
        didx[s][pl.ds(j * C, C)] = v >> 16
      pltpu.async_copy(pay_hbm.at[sidx[s]], payb[s], semg[s])
      pltpu.async_copy(adst_hbm.at[didx[s]], adstb[s], semg[s])

    def drain(s):
      pltpu.make_async_copy(pay_hbm.at[sidx[s]], payb[s], semg[s]).wait()
      pltpu.make_async_copy(adst_hbm.at[didx[s]], adstb[s], semg[s]).wait()

    def compute(s):
      pay_v, adst_v = payb[s], adstb[s]

      def edge(i, carry2):
        a = pay_v[i, pl.ds(D, C)] + adst_v[i, :]
        w = jnp.exp(jnp.maximum(a, 0.2 * a))
        pay_v[i, pl.ds(D, C)] = w
        for h in range(H):
          wb = lax.broadcast(w[h], (C,))
          pay_v[i, pl.ds(h * C, C)] = pay_v[i, pl.ds(h * C, C)] * wb
        return carry2

      lax.fori_loop(0, EDGE_BLK, edge, 0)

    def scat(s):
      pltpu.async_copy(payb[s], acc.at[didx[s]], sems[s], add=True)

    def scat_wait(s):
      pltpu.make_async_copy(payb[s], acc.at[didx[s]], sems[s]).wait()

    def step(b, cur, prev, first=False):
      drain(cur)
      compute(cur)
      if not first:
        scat_wait(prev)
      scat(cur)

      @pl.when(b + 2 < nblk)
      def _():
        bn = b + 2

        @pl.when(bn % IDX_CHUNK == 0)
        def _():
          refill(bn)

        fire(bn, prev)

    refill(0)
    fire(0, 0)
    fire(1, 1)
    step(0, 0, 2, first=True)
    step(1, 1, 0)

    def triple(t, carry):
      base = 2 + 3 * t
      step(base, 2, 1)
      step(base + 1, 0, 2)
      step(base + 2, 1, 0)
      return carry

    lax.fori_loop(0, (nblk - 2) // 3, triple, 0)
    for i in range((nblk - 2) % 3):
      b = nblk - (nblk - 2) % 3 + i
      step(b, b % 3, (b - 1) % 3)
    scat_wait((nblk - 1) % 3)

    plsc.subcore_barrier()
    pltpu.sync_copy(acc.at[pl.ds(r0, rpt)],
                    out_hbm.at[cid, pl.ds(r0, rpt)])

  return k(packed3, pay, adstt, zeros_tbl)


def _tc_final(acc0, acc1, pay, adstt, bias_row, sel):
  n = pay.shape[0]

  def body(a0, a1, payr, adr, br, selr, out_ref):
    num = a0[:, :D] + a1[:, :D]
    den8 = a0[:, D:D + H] + a1[:, D:D + H]
    aa = payr[:, D:D + H] + adr[:, :H]
    ws = jnp.exp(jnp.maximum(aa, 0.2 * aa))
    xw = payr[:, :D]
    ws128 = jnp.dot(ws, selr[...], preferred_element_type=jnp.float32)
    den128 = jnp.dot(den8 + ws, selr[...],
                     preferred_element_type=jnp.float32)
    out_ref[...] = (num + ws128 * xw) / (den128 + 1e-16) + br[...]

  return pl.pallas_call(
      body,
      grid=(n // ROW_BLK,),
      in_specs=[
          pl.BlockSpec((ROW_BLK, PAY_W), lambda i: (i, 0)),
          pl.BlockSpec((ROW_BLK, PAY_W), lambda i: (i, 0)),
          pl.BlockSpec((ROW_BLK, PAY_W), lambda i: (i, 0)),
          pl.BlockSpec((ROW_BLK, C), lambda i: (i, 0)),
          pl.BlockSpec((1, D), lambda i: (0, 0)),
          pl.BlockSpec((H, D), lambda i: (0, 0)),
      ],
      out_specs=pl.BlockSpec((ROW_BLK, D), lambda i: (i, 0)),
      out_shape=jax.ShapeDtypeStruct((n, D), jnp.float32),
  )(acc0, acc1, pay, adstt, bias_row, sel)


def kernel(x, edge_index, W, att_src, att_dst, bias):
  x = x.astype(jnp.float32)
  src = edge_index[0].astype(jnp.int32)
  dst = edge_index[1].astype(jnp.int32)

  eye = jnp.eye(H, dtype=jnp.float32)
  a_src_m = jnp.pad(
      (att_src.astype(jnp.float32)[:, :, None] * eye[:, None, :])
      .reshape(H * C, H), ((0, 0), (0, C - H)))
  a_dst_m = jnp.pad(
      (att_dst.astype(jnp.float32)[:, :, None] * eye[:, None, :])
      .reshape(H * C, H), ((0, 0), (0, C - H)))

  pay, adstt = _tc_prep(x, W.astype(jnp.float32), a_src_m, a_dst_m)
  n = x.shape[0]
  npad = -(-n // (NS * 8)) * (NS * 8)
  zeros_tbl = jnp.zeros((npad, PAY_W), jnp.float32)
  acc = _sc_edge(src, dst, pay, adstt, zeros_tbl)

  sel = jnp.repeat(eye, C, axis=1)
  return _tc_final(acc[0, :n], acc[1, :n], pay, adstt,
                   bias.astype(jnp.float32).reshape(1, D), sel)


# speedup vs baseline: 181.6617x; 1.4554x over previous
import functools

import jax
import jax.numpy as jnp
from jax import lax
from jax.experimental import pallas as pl
from jax.experimental.pallas import tpu as pltpu
from jax.experimental.pallas import tpu_sc as plsc

D = 128
H = 8
C = 16
PAY_W = D + C
NC = 2
NS = 16
NW = NC * NS
EDGE_BLK = 80
IDX_CHUNK = 5
ROW_BLK = 1000


def _tc_prep(x, w_mat, a_src_m, a_dst_m):
  n = x.shape[0]

  def body(x_ref, w_ref, ms_ref, md_ref, pay_ref, adst_ref):
    xw = jnp.dot(x_ref[...], w_ref[...], preferred_element_type=jnp.float32)
    asrc = jnp.dot(xw, ms_ref[...], preferred_element_type=jnp.float32)
    adstv = jnp.dot(xw, md_ref[...], preferred_element_type=jnp.float32)
    pay_ref[...] = jnp.concatenate([xw, asrc], axis=1)
    adst_ref[...] = adstv

  return pl.pallas_call(
      body,
      grid=(n // ROW_BLK,),
      in_specs=[
          pl.BlockSpec((ROW_BLK, D), lambda i: (i, 0)),
          pl.BlockSpec((D, D), lambda i: (0, 0)),
          pl.BlockSpec((D, C), lambda i: (0, 0)),
          pl.BlockSpec((D, C), lambda i: (0, 0)),
      ],
      out_specs=[
          pl.BlockSpec((ROW_BLK, PAY_W), lambda i: (i, 0)),
          pl.BlockSpec((ROW_BLK, C), lambda i: (i, 0)),
      ],
      out_shape=[
          jax.ShapeDtypeStruct((n, PAY_W), jnp.float32),
          jax.ShapeDtypeStruct((n, C), jnp.float32),
      ],
  )(x, w_mat, a_src_m, a_dst_m)


def _sc_edge(src, dst, pay, adstt):
  e = src.shape[0]
  epw = e // NW
  nblk = epw // EDGE_BLK
  n = pay.shape[0]
  npad = -(-n // (NS * 8)) * (NS * 8)
  rpt = npad // NS
  assert nblk % IDX_CHUNK == 0 and nblk >= 2

  packed3 = (src | (dst << 16)).reshape(NW, nblk, EDGE_BLK)

  mesh = plsc.VectorSubcoreMesh(
      core_axis_name="c", subcore_axis_name="s",
      num_cores=NC, num_subcores=NS)

  @functools.partial(
      pl.kernel,
      out_type=jax.ShapeDtypeStruct((NC, npad, PAY_W), jnp.float32),
      mesh=mesh,
      compiler_params=pltpu.CompilerParams(use_tc_tiling_on_sc=False),
      scratch_types=[
          pltpu.VMEM((2 * IDX_CHUNK, EDGE_BLK), jnp.int32),
          [pltpu.VMEM((EDGE_BLK,), jnp.int32) for _ in range(3)],
          [pltpu.VMEM((EDGE_BLK,), jnp.int32) for _ in range(3)],
          [pltpu.VMEM((EDGE_BLK, PAY_W), jnp.float32) for _ in range(3)],
          [pltpu.VMEM((EDGE_BLK, C), jnp.float32) for _ in range(3)],
          pltpu.VMEM_SHARED((npad, PAY_W), jnp.float32),
          [pltpu.SemaphoreType.DMA for _ in range(3)],
          [pltpu.SemaphoreType.DMA for _ in range(3)],
          pltpu.SemaphoreType.DMA,
      ],
  )
  def k(packed_hbm, pay_hbm, adst_hbm, out_hbm,
        pk, sidx, didx, payb, adstb, acc, semg, sems, semr):
    cid = lax.axis_index("c")
    sid = lax.axis_index("s")
    wid = sid * NC + cid

    @plsc.parallel_loop(0, EDGE_BLK)
    def zrow(i):
      for j in range(PAY_W // C):
        payb[0][i, pl.ds(j * C, C)] = jnp.zeros((C,), jnp.float32)

    r0 = sid * rpt
    for kk in range(rpt // EDGE_BLK):
      pltpu.sync_copy(payb[0], acc.at[pl.ds(r0 + kk * EDGE_BLK, EDGE_BLK)])
    if rpt % EDGE_BLK:
      pltpu.sync_copy(
          payb[0].at[pl.ds(0, rpt % EDGE_BLK)],
          acc.at[pl.ds(r0 + (rpt // EDGE_BLK) * EDGE_BLK, rpt % EDGE_BLK)])
    plsc.subcore_barrier()

    def refill(b):
      pltpu.async_copy(packed_hbm.at[wid, pl.ds(b, IDX_CHUNK)],
                       pk.at[pl.ds(b % (2 * IDX_CHUNK), IDX_CHUNK)], semr)

    def refill_wait(b):
      pltpu.make_async_copy(
          packed_hbm.at[wid, pl.ds(b, IDX_CHUNK)],
          pk.at[pl.ds(b % (2 * IDX_CHUNK), IDX_CHUNK)], semr).wait()

    def fire(b, s):
      bm = b % (2 * IDX_CHUNK)
      for j in range(EDGE_BLK // C):
        v = pk[bm, pl.ds(j * C, C)]
        sidx[s][pl.ds(j * C, C)] = v & 0xFFFF
        didx[s][pl.ds(j * C, C)] = v >> 16
      pltpu.async_copy(pay_hbm.at[sidx[s]], payb[s], semg[s])
      pltpu.async_copy(adst_hbm.at[didx[s]], adstb[s], semg[s])

    def drain(s):
      pltpu.make_async_copy(pay_hbm.at[sidx[s]], payb[s], semg[s]).wait()
      pltpu.make_async_copy(adst_hbm.at[didx[s]], adstb[s], semg[s]).wait()

    def compute(s):
      pay_v, adst_v = payb[s], adstb[s]

      @plsc.parallel_loop(0, EDGE_BLK, unroll=4)
      def edge(i):
        a = pay_v[i, pl.ds(D, C)] + adst_v[i, :]
        w = jnp.exp(jnp.maximum(a, 0.2 * a))
        pay_v[i, pl.ds(D, C)] = w
        for h in range(H):
          wb = lax.broadcast(w[h], (C,))
          pay_v[i, pl.ds(h * C, C)] = pay_v[i, pl.ds(h * C, C)] * wb

    def scat(s):
      pltpu.async_copy(payb[s], acc.at[didx[s]], sems[s], add=True)

    def scat_wait(s):
      pltpu.make_async_copy(payb[s], acc.at[didx[s]], sems[s]).wait()

    def step(b, cur, prev, first=False):
      drain(cur)
      compute(cur)
      if not first:
        scat_wait(prev)
      scat(cur)

      @pl.when(b + 2 < nblk)
      def _():
        bn = b + 2

        @pl.when(bn % IDX_CHUNK == 0)
        def _():
          refill_wait(bn)

          @pl.when(bn + IDX_CHUNK < nblk)
          def _():
            refill(bn + IDX_CHUNK)

        fire(bn, prev)

    refill(0)
    refill_wait(0)
    if nblk > IDX_CHUNK:
      refill(IDX_CHUNK)
    fire(0, 0)
    fire(1, 1)
    step(0, 0, 2, first=True)
    step(1, 1, 0)

    def triple(t, carry):
      base = 2 + 3 * t
      step(base, 2, 1)
      step(base + 1, 0, 2)
      step(base + 2, 1, 0)
      return carry

    lax.fori_loop(0, (nblk - 2) // 3, triple, 0)
    for i in range((nblk - 2) % 3):
      b = nblk - (nblk - 2) % 3 + i
      step(b, b % 3, (b - 1) % 3)
    scat_wait((nblk - 1) % 3)

    plsc.subcore_barrier()
    pltpu.sync_copy(acc.at[pl.ds(r0, rpt)],
                    out_hbm.at[cid, pl.ds(r0, rpt)])

  return k(packed3, pay, adstt)


def _tc_final(acc, pay, adstt, bias_row, sel):
  n = pay.shape[0]

  def body(ar, payr, adr, br, selr, out_ref):
    a0 = ar.at[0]
    a1 = ar.at[1]
    num = a0[:, :D] + a1[:, :D]
    den8 = a0[:, D:D + H] + a1[:, D:D + H]
    aa = payr[:, D:D + H] + adr[:, :H]
    ws = jnp.exp(jnp.maximum(aa, 0.2 * aa))
    xw = payr[:, :D]
    ws128 = jnp.dot(ws, selr[...], preferred_element_type=jnp.float32)
    den128 = jnp.dot(den8 + ws, selr[...],
                     preferred_element_type=jnp.float32)
    out_ref[...] = (num + ws128 * xw) / (den128 + 1e-16) + br[...]

  return pl.pallas_call(
      body,
      grid=(n // ROW_BLK,),
      in_specs=[
          pl.BlockSpec((2, ROW_BLK, PAY_W), lambda i: (0, i, 0)),
          pl.BlockSpec((ROW_BLK, PAY_W), lambda i: (i, 0)),
          pl.BlockSpec((ROW_BLK, C), lambda i: (i, 0)),
          pl.BlockSpec((1, D), lambda i: (0, 0)),
          pl.BlockSpec((H, D), lambda i: (0, 0)),
      ],
      out_specs=pl.BlockSpec((ROW_BLK, D), lambda i: (i, 0)),
      out_shape=jax.ShapeDtypeStruct((n, D), jnp.float32),
  )(acc, pay, adstt, bias_row, sel)


def kernel(x, edge_index, W, att_src, att_dst, bias):
  x = x.astype(jnp.float32)
  src = edge_index[0].astype(jnp.int32)
  dst = edge_index[1].astype(jnp.int32)

  eye = jnp.eye(H, dtype=jnp.float32)
  a_src_m = jnp.pad(
      (att_src.astype(jnp.float32)[:, :, None] * eye[:, None, :])
      .reshape(H * C, H), ((0, 0), (0, C - H)))
  a_dst_m = jnp.pad(
      (att_dst.astype(jnp.float32)[:, :, None] * eye[:, None, :])
      .reshape(H * C, H), ((0, 0), (0, C - H)))

  pay, adstt = _tc_prep(x, W.astype(jnp.float32), a_src_m, a_dst_m)
  acc = _sc_edge(src, dst, pay, adstt)

  sel = jnp.repeat(eye, C, axis=1)
  return _tc_final(acc, pay, adstt,
                   bias.astype(jnp.float32).reshape(1, D), sel)
